# baseline (speedup 1.0000x reference)
import functools

import jax
import jax.numpy as jnp
import numpy as np
from jax import lax
from jax.experimental import pallas as pl
from jax.experimental.pallas import tpu as pltpu
from jax.experimental.pallas import tpu_sc as plsc

THRESH = 0.6
MIN_KEPT = 65536

_BLK = 32768
_KEY_THRESH = int(np.float32(THRESH).view(np.int32))

_N = 1048576
_NTILE = 16
_NT = _N // _NTILE
_CH = 16384
_NCHUNK = _NT // _CH
_NB = 256
_NH = 4


def _pass1_body(pred_ref, tgt_ref, logp_ref, key_ref):
    x = pred_ref[0]
    t = tgt_ref[0]
    m = jnp.max(x, axis=0, keepdims=True)
    e = jnp.exp(x - m)
    cls = lax.broadcasted_iota(jnp.int32, x.shape, 0)
    sel = jnp.where(cls == t, x, 0.0)
    ones_row = jnp.ones((1, x.shape[0]), jnp.float32)
    dn = (((1,), (0,)), ((), ()))
    s = lax.dot_general(ones_row, e, dn,
                        preferred_element_type=jnp.float32)
    xt = lax.dot_general(ones_row, sel, dn,
                         preferred_element_type=jnp.float32)
    lse = m + jnp.log(s)
    logp = xt - lse
    p = jnp.exp(xt - m) / s
    key = lax.bitcast_convert_type(p, jnp.int32)
    logp_ref[0] = logp
    key_ref[0] = key


def _pass1(pred, tgt):
    b, c, s = pred.shape
    grid = (b, s // _BLK)
    return pl.pallas_call(
        _pass1_body,
        grid=grid,
        in_specs=[
            pl.BlockSpec((1, c, _BLK), lambda i, j: (i, 0, j)),
            pl.BlockSpec((1, 1, _BLK), lambda i, j: (i, 0, j)),
        ],
        out_specs=[
            pl.BlockSpec((1, 1, _BLK), lambda i, j: (i, 0, j)),
            pl.BlockSpec((1, 1, _BLK), lambda i, j: (i, 0, j)),
        ],
        out_shape=[
            jax.ShapeDtypeStruct((b, 1, s), jnp.float32),
            jax.ShapeDtypeStruct((b, 1, s), jnp.int32),
        ],
    )(pred, tgt.reshape(b, 1, s))


def _sc_body(keys_hbm, logp_hbm, out_hbm, keys_res, lbuf, h0, h1, h2, h3,
             hred, part, tots, csl, resv, sh_hist, sh_tot, sh_cs):
    tid = lax.axis_index("s")
    base = tid * _NT
    lane = lax.iota(jnp.int32, 16)
    zero16 = jnp.zeros((16,), jnp.int32)
    ones16 = jnp.ones((16,), jnp.int32)
    lane_off = lane * _NB
    hists = (h0, h1, h2, h3)

    pltpu.sync_copy(keys_hbm.at[pl.ds(base, _NT)], keys_res)

    k_rem = jnp.int32(MIN_KEPT)
    sel_prefix = jnp.int32(0)

    for pi in range(4):
        shift = 24 - 8 * pi

        @plsc.parallel_loop(0, _NB * 16 // 16, unroll=4)
        def _(j):
            for h in hists:
                h[pl.ds(j * 16, 16)] = zero16

        if pi == 0:
            @plsc.parallel_loop(0, _NT // (16 * _NH), unroll=4)
            def _(i, _sh=shift):
                for c in range(_NH):
                    kv = keys_res[pl.ds((i * _NH + c) * 16, 16)]
                    b_ = lax.shift_right_logical(kv, _sh) & (_NB - 1)
                    plsc.addupdate_scatter(hists[c], [lane_off + b_], ones16)
        else:
            @plsc.parallel_loop(0, _NT // (16 * _NH), unroll=4)
            def _(i, _sh=shift):
                for c in range(_NH):
                    kv = keys_res[pl.ds((i * _NH + c) * 16, 16)]
                    ok = lax.shift_right_logical(kv, _sh + 8) == sel_prefix
                    b_ = lax.shift_right_logical(kv, _sh) & (_NB - 1)
                    plsc.addupdate_scatter(hists[c], [lane_off + b_],
                                           ones16, mask=ok)

        def rbody(j, _):
            acc = zero16
            for h in hists:
                for l in range(16):
                    acc = acc + h[pl.ds(l * _NB + j * 16, 16)]
            hred[pl.ds(j * 16, 16)] = acc
            return 0

        lax.fori_loop(0, _NB // 16, rbody, 0)

        pltpu.sync_copy(hred, sh_hist.at[pl.ds(tid * _NB, _NB)])
        plsc.subcore_barrier()

        nb_per = _NB // _NTILE
        for l in range(_NTILE):
            pltpu.sync_copy(
                sh_hist.at[pl.ds(l * _NB + tid * nb_per, nb_per)],
                part.at[pl.ds(l * nb_per, nb_per)])

        acc = zero16
        for l in range(16):
            acc = acc + part[pl.ds(l * nb_per, nb_per)]
        hred[pl.ds(0, 16)] = acc
        pltpu.sync_copy(hred.at[pl.ds(0, nb_per)],
                        sh_tot.at[pl.ds(tid * nb_per, nb_per)])
        plsc.subcore_barrier()

        pltpu.sync_copy(sh_tot, tots)

        def sbody(j, carry, _k=k_rem):
            cnt, bin_sel, base_sel = carry
            v = tots[pl.ds(j * 16, 16)]
            cums = cnt + plsc.cumsum(v)
            tot = cnt + jnp.sum(v)
            found = (cnt < _k) & (tot >= _k)
            lane_idx = jnp.sum((cums < _k).astype(jnp.int32))
            b_ = j * 16 + lane_idx
            below = cnt + jnp.sum(jnp.where(lane < lane_idx, v, 0))
            bin_sel = jnp.where(found, b_, bin_sel)
            base_sel = jnp.where(found, below, base_sel)
            return (tot, bin_sel, base_sel)

        _, bin_sel, base_sel = lax.fori_loop(
            0, _NB // 16, sbody,
            (jnp.int32(0), jnp.int32(0), jnp.int32(0)))

        sel_prefix = (sel_prefix << 8) | bin_sel
        k_rem = k_rem - base_sel

    thr_key = jnp.maximum(sel_prefix, jnp.int32(_KEY_THRESH))

    cacc0 = zero16
    cacc1 = zero16
    sacc0 = jnp.zeros((16,), jnp.float32)
    sacc1 = jnp.zeros((16,), jnp.float32)
    for ci in range(_NCHUNK):
        pltpu.sync_copy(logp_hbm.at[pl.ds(base + ci * _CH, _CH)], lbuf)

        @plsc.parallel_loop(0, _CH // 32, unroll=4,
                            carry=(cacc0, sacc0, cacc1, sacc1))
        def facc(i, carry, _ci=ci):
            ca0, sa0, ca1, sa1 = carry
            koff = _ci * _CH + i * 32
            kv0 = keys_res[pl.ds(koff, 16)]
            lv0 = lbuf[pl.ds(i * 32, 16)]
            kv1 = keys_res[pl.ds(koff + 16, 16)]
            lv1 = lbuf[pl.ds(i * 32 + 16, 16)]
            m0 = kv0 <= thr_key
            m1 = kv1 <= thr_key
            ca0 = ca0 + jnp.where(m0, ones16, zero16)
            sa0 = sa0 + jnp.where(m0, lv0, 0.0)
            ca1 = ca1 + jnp.where(m1, ones16, zero16)
            sa1 = sa1 + jnp.where(m1, lv1, 0.0)
            return (ca0, sa0, ca1, sa1)

        cacc0, sacc0, cacc1, sacc1 = facc

    c_t = jnp.sum(cacc0 + cacc1).astype(jnp.float32)
    s_t = jnp.sum(sacc0 + sacc1)
    vec = jnp.where(lane == 0, c_t, 0.0) + jnp.where(lane == 1, s_t, 0.0)
    resv[...] = vec
    pltpu.sync_copy(resv, sh_cs.at[pl.ds(tid * 16, 16)])
    plsc.subcore_barrier()

    @pl.when(tid == 0)
    def _():
        pltpu.sync_copy(sh_cs, csl)
        acc = jnp.zeros((16,), jnp.float32)
        for l in range(16):
            acc = acc + csl[pl.ds(l * 16, 16)]
        zf = jnp.zeros((16,), jnp.float32)
        cntv = zf + jnp.sum(jnp.where(lane == 0, acc, 0.0))
        smv = zf + jnp.sum(jnp.where(lane == 1, acc, 0.0))
        resv[...] = -smv / jnp.maximum(cntv, 1.0)
        pltpu.sync_copy(resv, out_hbm)


@functools.partial(
    pl.kernel,
    out_type=jax.ShapeDtypeStruct((16,), jnp.float32),
    mesh=plsc.VectorSubcoreMesh(
        core_axis_name="c", subcore_axis_name="s", num_cores=1),
    compiler_params=pltpu.CompilerParams(needs_layout_passes=False),
    scratch_types=[
        pltpu.VMEM((_NT,), jnp.int32),
        pltpu.VMEM((_CH,), jnp.float32),
        pltpu.VMEM((_NB * 16,), jnp.int32),
        pltpu.VMEM((_NB * 16,), jnp.int32),
        pltpu.VMEM((_NB * 16,), jnp.int32),
        pltpu.VMEM((_NB * 16,), jnp.int32),
        pltpu.VMEM((_NB,), jnp.int32),
        pltpu.VMEM((_NB,), jnp.int32),
        pltpu.VMEM((_NB,), jnp.int32),
        pltpu.VMEM((256,), jnp.float32),
        pltpu.VMEM((16,), jnp.float32),
        pltpu.VMEM_SHARED((_NTILE * _NB,), jnp.int32),
        pltpu.VMEM_SHARED((_NB,), jnp.int32),
        pltpu.VMEM_SHARED((_NTILE * 16,), jnp.float32),
    ],
)
def _sc_select(keys_hbm, logp_hbm, out_hbm, *scratch):
    _sc_body(keys_hbm, logp_hbm, out_hbm, *scratch)


def _ohem_loss(pred, target):
    b, c, h, w = pred.shape
    s = h * w
    pred3 = pred.reshape(b, c, s)

    logp, keys = _pass1(pred3, target.reshape(b, s))
    out = _sc_select(keys.reshape(-1), logp.reshape(-1))
    return out[0]


def kernel(results, target):
    loss = jnp.float32(0.0)
    for i in range(results.shape[0]):
        loss = loss + _ohem_loss(results[i], target)
    return loss


# speedup vs baseline: 1.0415x; 1.0415x over previous
import functools

import jax
import jax.numpy as jnp
import numpy as np
from jax import lax
from jax.experimental import pallas as pl
from jax.experimental.pallas import tpu as pltpu
from jax.experimental.pallas import tpu_sc as plsc

THRESH = 0.6
MIN_KEPT = 65536

_BLK = 131072
_KEY_THRESH = int(np.float32(THRESH).view(np.int32))

_N = 1048576
_NTILE = 16
_NT = _N // _NTILE
_CH = 16384
_NCHUNK = _NT // _CH
_NB = 256
_NH = 4


def _pass1_body(pred_ref, tgt_ref, logp_ref, key_ref):
    x = pred_ref[0]
    t = tgt_ref[0]
    m = jnp.max(x, axis=0, keepdims=True)
    e = jnp.exp(x - m)
    cls = lax.broadcasted_iota(jnp.int32, x.shape, 0)
    sel = jnp.where(cls == t, x, 0.0)
    ones_row = jnp.ones((1, x.shape[0]), jnp.float32)
    dn = (((1,), (0,)), ((), ()))
    s = lax.dot_general(ones_row, e, dn,
                        preferred_element_type=jnp.float32)
    xt = lax.dot_general(ones_row, sel, dn,
                         preferred_element_type=jnp.float32)
    lse = m + jnp.log(s)
    logp = xt - lse
    p = jnp.exp(xt - m) / s
    key = lax.bitcast_convert_type(p, jnp.int32)
    logp_ref[0] = logp
    key_ref[0] = key


def _pass1(pred, tgt):
    b, c, s = pred.shape
    grid = (b, s // _BLK)
    return pl.pallas_call(
        _pass1_body,
        grid=grid,
        in_specs=[
            pl.BlockSpec((1, c, _BLK), lambda i, j: (i, 0, j)),
            pl.BlockSpec((1, 1, _BLK), lambda i, j: (i, 0, j)),
        ],
        out_specs=[
            pl.BlockSpec((1, 1, _BLK), lambda i, j: (i, 0, j)),
            pl.BlockSpec((1, 1, _BLK), lambda i, j: (i, 0, j)),
        ],
        out_shape=[
            jax.ShapeDtypeStruct((b, 1, s), jnp.float32),
            jax.ShapeDtypeStruct((b, 1, s), jnp.int32),
        ],
    )(pred, tgt.reshape(b, 1, s))


def _sc_body(keys_hbm, logp_hbm, out_hbm, keys_res, lbuf, h0, h1, h2, h3,
             hred, part, tots, csl, resv, sh_hist, sh_tot, sh_cs):
    tid = lax.axis_index("s")
    base = tid * _NT
    lane = lax.iota(jnp.int32, 16)
    zero16 = jnp.zeros((16,), jnp.int32)
    ones16 = jnp.ones((16,), jnp.int32)
    lane_off = lane * _NB
    hists = (h0, h1, h2, h3)

    pltpu.sync_copy(keys_hbm.at[pl.ds(base, _NT)], keys_res)

    k_rem = jnp.int32(MIN_KEPT)
    sel_prefix = jnp.int32(0)

    for pi in range(4):
        shift = 24 - 8 * pi

        @plsc.parallel_loop(0, _NB * 16 // 16, unroll=4)
        def _(j):
            for h in hists:
                h[pl.ds(j * 16, 16)] = zero16

        if pi == 0:
            @plsc.parallel_loop(0, _NT // (16 * _NH), unroll=4)
            def _(i, _sh=shift):
                for c in range(_NH):
                    kv = keys_res[pl.ds((i * _NH + c) * 16, 16)]
                    b_ = lax.shift_right_logical(kv, _sh) & (_NB - 1)
                    plsc.addupdate_scatter(hists[c], [lane_off + b_], ones16)
        else:
            @plsc.parallel_loop(0, _NT // (16 * _NH), unroll=4)
            def _(i, _sh=shift):
                for c in range(_NH):
                    kv = keys_res[pl.ds((i * _NH + c) * 16, 16)]
                    ok = lax.shift_right_logical(kv, _sh + 8) == sel_prefix
                    b_ = lax.shift_right_logical(kv, _sh) & (_NB - 1)
                    plsc.addupdate_scatter(hists[c], [lane_off + b_],
                                           ones16, mask=ok)

        def rbody(j, _):
            acc = zero16
            for h in hists:
                for l in range(16):
                    acc = acc + h[pl.ds(l * _NB + j * 16, 16)]
            hred[pl.ds(j * 16, 16)] = acc
            return 0

        lax.fori_loop(0, _NB // 16, rbody, 0)

        pltpu.sync_copy(hred, sh_hist.at[pl.ds(tid * _NB, _NB)])
        plsc.subcore_barrier()

        nb_per = _NB // _NTILE
        for l in range(_NTILE):
            pltpu.sync_copy(
                sh_hist.at[pl.ds(l * _NB + tid * nb_per, nb_per)],
                part.at[pl.ds(l * nb_per, nb_per)])

        acc = zero16
        for l in range(16):
            acc = acc + part[pl.ds(l * nb_per, nb_per)]
        hred[pl.ds(0, 16)] = acc
        pltpu.sync_copy(hred.at[pl.ds(0, nb_per)],
                        sh_tot.at[pl.ds(tid * nb_per, nb_per)])
        plsc.subcore_barrier()

        pltpu.sync_copy(sh_tot, tots)

        def sbody(j, carry, _k=k_rem):
            cnt, bin_sel, base_sel = carry
            v = tots[pl.ds(j * 16, 16)]
            cums = cnt + plsc.cumsum(v)
            tot = cnt + jnp.sum(v)
            found = (cnt < _k) & (tot >= _k)
            lane_idx = jnp.sum((cums < _k).astype(jnp.int32))
            b_ = j * 16 + lane_idx
            below = cnt + jnp.sum(jnp.where(lane < lane_idx, v, 0))
            bin_sel = jnp.where(found, b_, bin_sel)
            base_sel = jnp.where(found, below, base_sel)
            return (tot, bin_sel, base_sel)

        _, bin_sel, base_sel = lax.fori_loop(
            0, _NB // 16, sbody,
            (jnp.int32(0), jnp.int32(0), jnp.int32(0)))

        sel_prefix = (sel_prefix << 8) | bin_sel
        k_rem = k_rem - base_sel

    thr_key = jnp.maximum(sel_prefix, jnp.int32(_KEY_THRESH))

    cacc0 = zero16
    cacc1 = zero16
    sacc0 = jnp.zeros((16,), jnp.float32)
    sacc1 = jnp.zeros((16,), jnp.float32)
    for ci in range(_NCHUNK):
        pltpu.sync_copy(logp_hbm.at[pl.ds(base + ci * _CH, _CH)], lbuf)

        @plsc.parallel_loop(0, _CH // 32, unroll=4,
                            carry=(cacc0, sacc0, cacc1, sacc1))
        def facc(i, carry, _ci=ci):
            ca0, sa0, ca1, sa1 = carry
            koff = _ci * _CH + i * 32
            kv0 = keys_res[pl.ds(koff, 16)]
            lv0 = lbuf[pl.ds(i * 32, 16)]
            kv1 = keys_res[pl.ds(koff + 16, 16)]
            lv1 = lbuf[pl.ds(i * 32 + 16, 16)]
            m0 = kv0 <= thr_key
            m1 = kv1 <= thr_key
            ca0 = ca0 + jnp.where(m0, ones16, zero16)
            sa0 = sa0 + jnp.where(m0, lv0, 0.0)
            ca1 = ca1 + jnp.where(m1, ones16, zero16)
            sa1 = sa1 + jnp.where(m1, lv1, 0.0)
            return (ca0, sa0, ca1, sa1)

        cacc0, sacc0, cacc1, sacc1 = facc

    c_t = jnp.sum(cacc0 + cacc1).astype(jnp.float32)
    s_t = jnp.sum(sacc0 + sacc1)
    vec = jnp.where(lane == 0, c_t, 0.0) + jnp.where(lane == 1, s_t, 0.0)
    resv[...] = vec
    pltpu.sync_copy(resv, sh_cs.at[pl.ds(tid * 16, 16)])
    plsc.subcore_barrier()

    @pl.when(tid == 0)
    def _():
        pltpu.sync_copy(sh_cs, csl)
        acc = jnp.zeros((16,), jnp.float32)
        for l in range(16):
            acc = acc + csl[pl.ds(l * 16, 16)]
        zf = jnp.zeros((16,), jnp.float32)
        cntv = zf + jnp.sum(jnp.where(lane == 0, acc, 0.0))
        smv = zf + jnp.sum(jnp.where(lane == 1, acc, 0.0))
        resv[...] = -smv / jnp.maximum(cntv, 1.0)
        pltpu.sync_copy(resv, out_hbm)


@functools.partial(
    pl.kernel,
    out_type=jax.ShapeDtypeStruct((16,), jnp.float32),
    mesh=plsc.VectorSubcoreMesh(
        core_axis_name="c", subcore_axis_name="s", num_cores=1),
    compiler_params=pltpu.CompilerParams(needs_layout_passes=False),
    scratch_types=[
        pltpu.VMEM((_NT,), jnp.int32),
        pltpu.VMEM((_CH,), jnp.float32),
        pltpu.VMEM((_NB * 16,), jnp.int32),
        pltpu.VMEM((_NB * 16,), jnp.int32),
        pltpu.VMEM((_NB * 16,), jnp.int32),
        pltpu.VMEM((_NB * 16,), jnp.int32),
        pltpu.VMEM((_NB,), jnp.int32),
        pltpu.VMEM((_NB,), jnp.int32),
        pltpu.VMEM((_NB,), jnp.int32),
        pltpu.VMEM((256,), jnp.float32),
        pltpu.VMEM((16,), jnp.float32),
        pltpu.VMEM_SHARED((_NTILE * _NB,), jnp.int32),
        pltpu.VMEM_SHARED((_NB,), jnp.int32),
        pltpu.VMEM_SHARED((_NTILE * 16,), jnp.float32),
    ],
)
def _sc_select(keys_hbm, logp_hbm, out_hbm, *scratch):
    _sc_body(keys_hbm, logp_hbm, out_hbm, *scratch)


def _ohem_loss(pred, target):
    b, c, h, w = pred.shape
    s = h * w
    pred3 = pred.reshape(b, c, s)

    logp, keys = _pass1(pred3, target.reshape(b, s))
    out = _sc_select(keys.reshape(-1), logp.reshape(-1))
    return out[0]


def kernel(results, target):
    loss = jnp.float32(0.0)
    for i in range(results.shape[0]):
        loss = loss + _ohem_loss(results[i], target)
    return loss
